# baseline (speedup 1.0000x reference)
import functools

import jax
import jax.numpy as jnp
from jax import lax
from jax.experimental import pallas as pl
from jax.experimental.pallas import tpu as pltpu

B = 4
C = 256
H8 = 32
W8 = 64
Q = H8 * W8
R = 4

_OFFS = []
for _y in range(-R, R + 1):
    for _x in range(abs(_y) - R, R - abs(_y) + 1):
        _OFFS.append((_x, _y))
L = len(_OFFS)


def _weight_vectors(pb, scale, n, npix):
    offs = (lax.broadcasted_iota(jnp.int32, (9, 1), 0) - R).astype(jnp.float32)
    p = pb[None, :] + offs * scale
    f = jnp.floor(pb - 4.0 * scale)
    ws = jnp.clip(f, 0.0, float(n - 3)).astype(jnp.int32)
    p0 = jnp.floor(p)
    w1 = p - p0
    w0 = 1.0 - w1
    t0 = p0.astype(jnp.int32)
    t1 = t0 + 1
    v0 = ((p0 >= 0.0) & (p0 <= float(n - 1))).astype(jnp.float32)
    v1 = ((p0 + 1.0 >= 0.0) & (p0 + 1.0 <= float(n - 1))).astype(jnp.float32)
    d0 = t0 - ws[None, :]
    d1 = t1 - ws[None, :]
    w3 = []
    for d in range(3):
        w3.append(jnp.where(d0 == d, w0 * v0, 0.0) + jnp.where(d1 == d, w1 * v1, 0.0))
    return ws, jnp.stack(w3, axis=1)


QT = 128


def _yselect(vol_ref, ys, hk, wk):
    rows = [[], [], []]
    for qt in range(Q // QT):
        qs = slice(qt * QT, (qt + 1) * QT)
        ys_t = ys[qs]

        def body(y, accs, ys_t=ys_t, qs=qs):
            a0, a1, a2 = accs
            v = vol_ref[pl.ds(y, 1), :, qs].reshape(wk, QT)
            m0 = (ys_t == y).astype(jnp.float32)[None, :]
            m1 = (ys_t == y - 1).astype(jnp.float32)[None, :]
            m2 = (ys_t == y - 2).astype(jnp.float32)[None, :]
            return a0 + v * m0, a1 + v * m1, a2 + v * m2

        z = jnp.zeros((wk, QT), jnp.float32)
        accs = lax.fori_loop(0, hk, body, (z, z, z))
        for dy in range(3):
            rows[dy].append(accs[dy])
    return [jnp.concatenate(r, axis=1) for r in rows]


def _lookup_body(f2t_ref, f1_ref, flow_ref, out_ref,
                 v0_ref, v1_ref, v2_ref, v3_ref):
    f2t = f2t_ref[0]
    f1 = f1_ref[0]
    fy = flow_ref[0]
    fx = flow_ref[1]

    corrT = jnp.dot(f2t.astype(jnp.bfloat16), f1.astype(jnp.bfloat16),
                    preferred_element_type=jnp.float32) * (1.0 / 16.0)

    qi = lax.broadcasted_iota(jnp.int32, (Q,), 0)
    jj = (qi % W8).astype(jnp.float32)
    ii = (qi // W8).astype(jnp.float32)

    vol = corrT.reshape(H8, W8, Q)
    vrefs = [v0_ref, v1_ref, v2_ref, v3_ref]
    outs = []
    for k in range(4):
        hk = H8 >> k
        wk = W8 >> k
        if k > 0:
            a = vol.reshape(hk, 2, wk, 2, Q)
            vol = (a[:, 0, :, 0] + a[:, 0, :, 1] + a[:, 1, :, 0] + a[:, 1, :, 1]) * 0.25
        vrefs[k][...] = vol

        sx = float((1 << k) * (wk - 1)) / 512.0
        sy = float((1 << k) * (hk - 1)) / 256.0
        pbx = (jj + fx) * (float(wk - 1) / 512.0)
        pby = (ii + fy) * (float(hk - 1) / 256.0)
        xs, wx3 = _weight_vectors(pbx, sx, wk, Q)
        ys, wy3 = _weight_vectors(pby, sy, hk, Q)

        rows = _yselect(vrefs[k], ys, hk, wk)

        lxi = lax.broadcasted_iota(jnp.int32, (wk, Q), 0)
        win = []
        for dy in range(3):
            row_dy = []
            for dx in range(3):
                m = (lxi == (xs + dx)[None, :]).astype(jnp.float32)
                row_dy.append(jnp.sum(rows[dy] * m, axis=0))
            win.append(row_dy)

        t = [[None] * 9 for _ in range(3)]
        for dy in range(3):
            for xo in range(9):
                t[dy][xo] = (wx3[xo, 0] * win[dy][0]
                             + wx3[xo, 1] * win[dy][1]
                             + wx3[xo, 2] * win[dy][2])

        lvl = []
        for (xo, yo) in _OFFS:
            v = (wy3[yo + 4, 0] * t[0][xo + 4]
                 + wy3[yo + 4, 1] * t[1][xo + 4]
                 + wy3[yo + 4, 2] * t[2][xo + 4])
            lvl.append(v)
        outs.append(jnp.stack(lvl, axis=0))

    out_ref[0] = jnp.stack(outs, axis=1)


@jax.jit
def kernel(feat1, feat2, curr_flow):
    f1 = feat1.reshape(B, C, Q)
    f2t = feat2.reshape(B, C, Q).transpose(0, 2, 1)
    flow = curr_flow.reshape(2, Q)

    out = pl.pallas_call(
        _lookup_body,
        grid=(B,),
        in_specs=[
            pl.BlockSpec((1, Q, C), lambda b: (b, 0, 0)),
            pl.BlockSpec((1, C, Q), lambda b: (b, 0, 0)),
            pl.BlockSpec((2, Q), lambda b: (0, 0)),
        ],
        out_specs=pl.BlockSpec((1, L, 4, Q), lambda b: (b, 0, 0, 0)),
        out_shape=jax.ShapeDtypeStruct((B, L, 4, Q), jnp.float32),
        scratch_shapes=[
            pltpu.VMEM((H8 >> k, W8 >> k, Q), jnp.float32) for k in range(4)
        ],
    )(f2t, f1, flow)
    return out.reshape(B, L, 4, H8, W8)


# speedup vs baseline: 1.3491x; 1.3491x over previous
import jax
import jax.numpy as jnp
from jax import lax
from jax.experimental import pallas as pl

B = 4
C = 256
H8 = 32
W8 = 64
Q = H8 * W8
R = 4

_OFFS = []
for _y in range(-R, R + 1):
    for _x in range(abs(_y) - R, R - abs(_y) + 1):
        _OFFS.append((_x, _y))
L = len(_OFFS)

_XPAR = [((float((1 << k) * ((W8 >> k) - 1)) / 512.0), W8 >> k) for k in range(4)]
_YPAR = [((float((1 << k) * ((H8 >> k) - 1)) / 256.0), H8 >> k) for k in range(4)]


def _all_weight_vectors(pbs):
    offs = (lax.broadcasted_iota(jnp.int32, (9, 1), 0) - R).astype(jnp.float32)
    prows = []
    ws_list = []
    nmax_rows = []
    for pb, s, n in pbs:
        prows.append(pb[None, :] + offs * s)
        f = jnp.floor(pb - 4.0 * s)
        ws_list.append(jnp.clip(f, 0.0, float(n - 3)).astype(jnp.int32))
        nmax_rows.append(jnp.full((9, Q), float(n - 1), jnp.float32))
    P = jnp.concatenate(prows, axis=0)
    NM = jnp.concatenate(nmax_rows, axis=0)
    WS = jnp.concatenate(
        [jnp.broadcast_to(w[None, :], (9, Q)) for w in ws_list], axis=0)
    p0 = jnp.floor(P)
    w1 = P - p0
    m0 = (1.0 - w1) * ((p0 >= 0.0) & (p0 <= NM)).astype(jnp.float32)
    m1 = w1 * ((p0 + 1.0 >= 0.0) & (p0 + 1.0 <= NM)).astype(jnp.float32)
    d0 = p0.astype(jnp.int32) - WS
    w3 = [jnp.where(d0 == d, m0, 0.0) + jnp.where(d0 == d - 1, m1, 0.0)
          for d in range(3)]
    W3 = jnp.stack(w3, axis=1)
    return ws_list, [W3[9 * i:9 * (i + 1)] for i in range(8)]


def _lookup_body(f2t_ref, f1_ref, flow_ref, out_ref):
    f2t = f2t_ref[0]
    f1 = f1_ref[0]
    fy = flow_ref[0]
    fx = flow_ref[1]

    corrT = jnp.dot(f2t.astype(jnp.bfloat16), f1.astype(jnp.bfloat16),
                    preferred_element_type=jnp.float32) * (1.0 / 16.0)

    qi = lax.broadcasted_iota(jnp.int32, (Q,), 0)
    jj = (qi % W8).astype(jnp.float32)
    ii = (qi // W8).astype(jnp.float32)

    pbs = ([((jj + fx) * (float((W8 >> k) - 1) / 512.0),) + _XPAR[k]
            for k in range(4)]
           + [((ii + fy) * (float((H8 >> k) - 1) / 256.0),) + _YPAR[k]
              for k in range(4)])
    starts, w3s = _all_weight_vectors(pbs)

    vol = corrT.reshape(H8, W8, Q)
    outs = []
    for k in range(4):
        hk = H8 >> k
        wk = W8 >> k
        if k > 0:
            a = vol.reshape(hk, 2, wk, 2, Q)
            vol = (a[:, 0, :, 0] + a[:, 0, :, 1] + a[:, 1, :, 0] + a[:, 1, :, 1]) * 0.25

        xs, wx3 = starts[k], w3s[k]
        ys, wy3 = starts[4 + k], w3s[4 + k]

        masks = [(ys == y).astype(jnp.float32)[None, :] for y in range(hk - 2)]
        rows = []
        for dy in range(3):
            acc = jnp.zeros((wk, Q), jnp.float32)
            for y in range(dy, hk - 2 + dy):
                acc = acc + vol[y] * masks[y - dy]
            rows.append(acc)

        lxi = lax.broadcasted_iota(jnp.int32, (wk, Q), 0)
        xmasks = [(lxi == (xs + dx)[None, :]).astype(jnp.float32)
                  for dx in range(3)]
        win = [[jnp.sum(rows[dy] * xmasks[dx], axis=0) for dx in range(3)]
               for dy in range(3)]

        winarr = jnp.stack([jnp.stack(w, axis=0) for w in win], axis=0)
        t = jnp.sum(winarr[:, None, :, :] * wx3[None, :, :, :], axis=2)

        lvl = []
        for yo in range(-R, R + 1):
            a0, a1 = abs(yo), 9 - abs(yo)
            g = (wy3[yo + R, 0][None, :] * t[0, a0:a1]
                 + wy3[yo + R, 1][None, :] * t[1, a0:a1]
                 + wy3[yo + R, 2][None, :] * t[2, a0:a1])
            lvl.append(g)
        outs.append(jnp.concatenate(lvl, axis=0))

    out_ref[0] = jnp.stack(outs, axis=1)


@jax.jit
def kernel(feat1, feat2, curr_flow):
    f1 = feat1.reshape(B, C, Q)
    f2t = feat2.reshape(B, C, Q).transpose(0, 2, 1)
    flow = curr_flow.reshape(2, Q)

    out = pl.pallas_call(
        _lookup_body,
        grid=(B,),
        in_specs=[
            pl.BlockSpec((1, Q, C), lambda b: (b, 0, 0)),
            pl.BlockSpec((1, C, Q), lambda b: (b, 0, 0)),
            pl.BlockSpec((2, Q), lambda b: (0, 0)),
        ],
        out_specs=pl.BlockSpec((1, L, 4, Q), lambda b: (b, 0, 0, 0)),
        out_shape=jax.ShapeDtypeStruct((B, L, 4, Q), jnp.float32),
    )(f2t, f1, flow)
    return out.reshape(B, L, 4, H8, W8)


# speedup vs baseline: 1.4590x; 1.0814x over previous
import jax
import jax.numpy as jnp
from jax import lax
from jax.experimental import pallas as pl

B = 4
C = 256
H8 = 32
W8 = 64
Q = H8 * W8
R = 4

_OFFS = []
for _y in range(-R, R + 1):
    for _x in range(abs(_y) - R, R - abs(_y) + 1):
        _OFFS.append((_x, _y))
L = len(_OFFS)

_XPAR = [((float((1 << k) * ((W8 >> k) - 1)) / 512.0), W8 >> k) for k in range(4)]
_YPAR = [((float((1 << k) * ((H8 >> k) - 1)) / 256.0), H8 >> k) for k in range(4)]


def _all_weight_vectors(pbs):
    offs = (lax.broadcasted_iota(jnp.int32, (9, 1), 0) - R).astype(jnp.float32)
    prows = []
    ws_list = []
    nmax_rows = []
    for pb, s, n in pbs:
        prows.append(pb[None, :] + offs * s)
        f = jnp.floor(pb - 4.0 * s)
        ws_list.append(jnp.clip(f, 0.0, float(n - 3)).astype(jnp.int32))
        nmax_rows.append(jnp.full((9, Q), float(n - 1), jnp.float32))
    P = jnp.concatenate(prows, axis=0)
    NM = jnp.concatenate(nmax_rows, axis=0)
    WS = jnp.concatenate(
        [jnp.broadcast_to(w[None, :], (9, Q)) for w in ws_list], axis=0)
    p0 = jnp.floor(P)
    w1 = P - p0
    m0 = (1.0 - w1) * ((p0 >= 0.0) & (p0 <= NM)).astype(jnp.float32)
    m1 = w1 * ((p0 + 1.0 >= 0.0) & (p0 + 1.0 <= NM)).astype(jnp.float32)
    d0 = p0.astype(jnp.int32) - WS
    w3 = [jnp.where(d0 == d, m0, 0.0) + jnp.where(d0 == d - 1, m1, 0.0)
          for d in range(3)]
    W3 = jnp.stack(w3, axis=1)
    return ws_list, [W3[9 * i:9 * (i + 1)] for i in range(8)]


def _lookup_body(f2t_ref, f1_ref, flow_ref, out_ref):
    f2t = f2t_ref[0]
    f1 = f1_ref[0]
    fy = flow_ref[0]
    fx = flow_ref[1]

    corrT = (jnp.dot(f2t.astype(jnp.bfloat16), f1.astype(jnp.bfloat16),
                     preferred_element_type=jnp.float32)
             * (1.0 / 16.0)).astype(jnp.bfloat16)

    qi = lax.broadcasted_iota(jnp.int32, (Q,), 0)
    jj = (qi % W8).astype(jnp.float32)
    ii = (qi // W8).astype(jnp.float32)

    pbs = ([((jj + fx) * (float((W8 >> k) - 1) / 512.0),) + _XPAR[k]
            for k in range(4)]
           + [((ii + fy) * (float((H8 >> k) - 1) / 256.0),) + _YPAR[k]
              for k in range(4)])
    starts, w3s = _all_weight_vectors(pbs)

    vol = corrT.reshape(H8, W8, Q)
    outs = []
    for k in range(4):
        hk = H8 >> k
        wk = W8 >> k
        if k > 0:
            a = vol.reshape(hk, 2, wk, 2, Q)
            vol = (a[:, 0, :, 0] + a[:, 0, :, 1] + a[:, 1, :, 0] + a[:, 1, :, 1]) * 0.25

        xs, wx3 = starts[k], w3s[k]
        ys, wy3 = starts[4 + k], w3s[4 + k]

        masks = [(ys == y).astype(jnp.bfloat16)[None, :] for y in range(hk - 2)]
        rows = []
        for dy in range(3):
            acc = jnp.zeros((wk, Q), jnp.bfloat16)
            for y in range(dy, hk - 2 + dy):
                acc = acc + vol[y] * masks[y - dy]
            rows.append(acc)

        lxi = lax.broadcasted_iota(jnp.int32, (wk, Q), 0)
        xmasks = [(lxi == (xs + dx)[None, :]).astype(jnp.bfloat16)
                  for dx in range(3)]
        win = [[jnp.sum(rows[dy] * xmasks[dx], axis=0) for dx in range(3)]
               for dy in range(3)]

        winarr = jnp.stack([jnp.stack(w, axis=0) for w in win],
                           axis=0).astype(jnp.float32)
        t = jnp.sum(winarr[:, None, :, :] * wx3[None, :, :, :], axis=2)

        lvl = []
        for yo in range(-R, R + 1):
            a0, a1 = abs(yo), 9 - abs(yo)
            g = (wy3[yo + R, 0][None, :] * t[0, a0:a1]
                 + wy3[yo + R, 1][None, :] * t[1, a0:a1]
                 + wy3[yo + R, 2][None, :] * t[2, a0:a1])
            lvl.append(g)
        outs.append(jnp.concatenate(lvl, axis=0))

    out_ref[0] = jnp.stack(outs, axis=1)


@jax.jit
def kernel(feat1, feat2, curr_flow):
    f1 = feat1.reshape(B, C, Q)
    f2t = feat2.reshape(B, C, Q).transpose(0, 2, 1)
    flow = curr_flow.reshape(2, Q)

    out = pl.pallas_call(
        _lookup_body,
        grid=(B,),
        in_specs=[
            pl.BlockSpec((1, Q, C), lambda b: (b, 0, 0)),
            pl.BlockSpec((1, C, Q), lambda b: (b, 0, 0)),
            pl.BlockSpec((2, Q), lambda b: (0, 0)),
        ],
        out_specs=pl.BlockSpec((1, L, 4, Q), lambda b: (b, 0, 0, 0)),
        out_shape=jax.ShapeDtypeStruct((B, L, 4, Q), jnp.float32),
    )(f2t, f1, flow)
    return out.reshape(B, L, 4, H8, W8)


# speedup vs baseline: 1.4683x; 1.0064x over previous
import jax
import jax.numpy as jnp
from jax import lax
from jax.experimental import pallas as pl

B = 4
C = 256
H8 = 32
W8 = 64
Q = H8 * W8
R = 4

_OFFS = []
for _y in range(-R, R + 1):
    for _x in range(abs(_y) - R, R - abs(_y) + 1):
        _OFFS.append((_x, _y))
L = len(_OFFS)

_XPAR = [((float((1 << k) * ((W8 >> k) - 1)) / 512.0), W8 >> k) for k in range(4)]
_YPAR = [((float((1 << k) * ((H8 >> k) - 1)) / 256.0), H8 >> k) for k in range(4)]


def _all_weight_vectors(pbs):
    offs = (lax.broadcasted_iota(jnp.int32, (9, 1), 0) - R).astype(jnp.float32)
    prows = []
    ws_list = []
    nmax_rows = []
    for pb, s, n in pbs:
        prows.append(pb[None, :] + offs * s)
        f = jnp.floor(pb - 4.0 * s)
        ws_list.append(jnp.clip(f, 0.0, float(n - 3)).astype(jnp.int32))
        nmax_rows.append(jnp.full((9, Q), float(n - 1), jnp.float32))
    P = jnp.concatenate(prows, axis=0)
    NM = jnp.concatenate(nmax_rows, axis=0)
    WS = jnp.concatenate(
        [jnp.broadcast_to(w[None, :], (9, Q)) for w in ws_list], axis=0)
    p0 = jnp.floor(P)
    w1 = P - p0
    m0 = (1.0 - w1) * ((p0 >= 0.0) & (p0 <= NM)).astype(jnp.float32)
    m1 = w1 * ((p0 + 1.0 >= 0.0) & (p0 + 1.0 <= NM)).astype(jnp.float32)
    d0 = p0.astype(jnp.int32) - WS
    w3 = [jnp.where(d0 == d, m0, 0.0) + jnp.where(d0 == d - 1, m1, 0.0)
          for d in range(3)]
    W3 = jnp.stack(w3, axis=1)
    return ws_list, [W3[9 * i:9 * (i + 1)] for i in range(8)]


def _lookup_body(f2t_ref, f1_ref, flow_ref, out_ref):
    fy = flow_ref[0]
    fx = flow_ref[1]

    qi = lax.broadcasted_iota(jnp.int32, (Q,), 0)
    jj = (qi % W8).astype(jnp.float32)
    ii = (qi // W8).astype(jnp.float32)

    pbs = ([((jj + fx) * (float((W8 >> k) - 1) / 512.0),) + _XPAR[k]
            for k in range(4)]
           + [((ii + fy) * (float((H8 >> k) - 1) / 256.0),) + _YPAR[k]
              for k in range(4)])
    starts, w3s = _all_weight_vectors(pbs)

    for b in range(B):
        _one_batch(f2t_ref[b], f1_ref[b], starts, w3s, out_ref, b)


def _one_batch(f2t, f1, starts, w3s, out_ref, b):
    corrT = (jnp.dot(f2t, f1, preferred_element_type=jnp.float32)
             * (1.0 / 16.0)).astype(jnp.bfloat16)

    vol = corrT.reshape(H8, W8, Q)
    outs = []
    for k in range(4):
        hk = H8 >> k
        wk = W8 >> k
        if k > 0:
            a = vol.reshape(hk, 2, wk, 2, Q)
            vol = (a[:, 0, :, 0] + a[:, 0, :, 1] + a[:, 1, :, 0] + a[:, 1, :, 1]) * 0.25

        xs, wx3 = starts[k], w3s[k]
        ys, wy3 = starts[4 + k], w3s[4 + k]

        masks = [(ys == y).astype(jnp.bfloat16)[None, :] for y in range(hk - 2)]
        rows = []
        for dy in range(3):
            acc = jnp.zeros((wk, Q), jnp.bfloat16)
            for y in range(dy, hk - 2 + dy):
                acc = acc + vol[y] * masks[y - dy]
            rows.append(acc)

        lxi = lax.broadcasted_iota(jnp.int32, (wk, Q), 0)
        xmasks = [(lxi == (xs + dx)[None, :]).astype(jnp.bfloat16)
                  for dx in range(3)]
        win = [[jnp.sum(rows[dy] * xmasks[dx], axis=0) for dx in range(3)]
               for dy in range(3)]

        winarr = jnp.stack([jnp.stack(w, axis=0) for w in win],
                           axis=0).astype(jnp.float32)
        t = jnp.sum(winarr[:, None, :, :] * wx3[None, :, :, :], axis=2)

        lvl = []
        for yo in range(-R, R + 1):
            a0, a1 = abs(yo), 9 - abs(yo)
            g = (wy3[yo + R, 0][None, :] * t[0, a0:a1]
                 + wy3[yo + R, 1][None, :] * t[1, a0:a1]
                 + wy3[yo + R, 2][None, :] * t[2, a0:a1])
            lvl.append(g)
        outs.append(jnp.concatenate(lvl, axis=0))

    out_ref[b] = jnp.stack(outs, axis=1)


@jax.jit
def kernel(feat1, feat2, curr_flow):
    f1 = feat1.reshape(B, C, Q).astype(jnp.bfloat16)
    f2t = feat2.reshape(B, C, Q).transpose(0, 2, 1).astype(jnp.bfloat16)
    flow = curr_flow.reshape(2, Q)

    out = pl.pallas_call(
        _lookup_body,
        out_shape=jax.ShapeDtypeStruct((B, L, 4, Q), jnp.float32),
    )(f2t, f1, flow)
    return out.reshape(B, L, 4, H8, W8)


# speedup vs baseline: 1.5509x; 1.0563x over previous
import jax
import jax.numpy as jnp
from jax import lax
from jax.experimental import pallas as pl

B = 4
C = 256
H8 = 32
W8 = 64
Q = H8 * W8
R = 4

_OFFS = []
for _y in range(-R, R + 1):
    for _x in range(abs(_y) - R, R - abs(_y) + 1):
        _OFFS.append((_x, _y))
L = len(_OFFS)

_XPAR = [((float((1 << k) * ((W8 >> k) - 1)) / 512.0), W8 >> k) for k in range(4)]
_YPAR = [((float((1 << k) * ((H8 >> k) - 1)) / 256.0), H8 >> k) for k in range(4)]


def _all_weight_vectors(pbs):
    offs = (lax.broadcasted_iota(jnp.int32, (9, 1), 0) - R).astype(jnp.float32)
    prows = []
    ws_list = []
    nmax_rows = []
    for pb, s, n in pbs:
        prows.append(pb[None, :] + offs * s)
        f = jnp.floor(pb - 4.0 * s)
        ws_list.append(jnp.clip(f, 0.0, float(n - 3)).astype(jnp.int32))
        nmax_rows.append(jnp.full((9, Q), float(n - 1), jnp.float32))
    P = jnp.concatenate(prows, axis=0)
    NM = jnp.concatenate(nmax_rows, axis=0)
    WS = jnp.concatenate(
        [jnp.broadcast_to(w[None, :], (9, Q)) for w in ws_list], axis=0)
    p0 = jnp.floor(P)
    w1 = P - p0
    m0 = (1.0 - w1) * ((p0 >= 0.0) & (p0 <= NM)).astype(jnp.float32)
    m1 = w1 * ((p0 + 1.0 >= 0.0) & (p0 + 1.0 <= NM)).astype(jnp.float32)
    d0 = p0.astype(jnp.int32) - WS
    w3 = [jnp.where(d0 == d, m0, 0.0) + jnp.where(d0 == d - 1, m1, 0.0)
          for d in range(3)]
    W3 = jnp.stack(w3, axis=1)
    return ws_list, [W3[9 * i:9 * (i + 1)] for i in range(8)]


def _lookup_body(f2t_ref, f1_ref, flow_ref, out_ref):
    fy = flow_ref[0]
    fx = flow_ref[1]

    qi = lax.broadcasted_iota(jnp.int32, (Q,), 0)
    jj = (qi % W8).astype(jnp.float32)
    ii = (qi // W8).astype(jnp.float32)

    pbs = ([((jj + fx) * (float((W8 >> k) - 1) / 512.0),) + _XPAR[k]
            for k in range(4)]
           + [((ii + fy) * (float((H8 >> k) - 1) / 256.0),) + _YPAR[k]
              for k in range(4)])
    starts, w3s = _all_weight_vectors(pbs)

    for b in range(B):
        _one_batch(f2t_ref[b], f1_ref[b], starts, w3s, out_ref, b)


def _one_batch(f2t, f1, starts, w3s, out_ref, b):
    corrT = (jnp.dot(f2t, f1, preferred_element_type=jnp.float32)
             * (1.0 / 16.0)).astype(jnp.bfloat16)

    vol = corrT.reshape(H8, W8, Q)
    outs = []
    for k in range(4):
        hk = H8 >> k
        wk = W8 >> k
        if k > 0:
            a = vol.reshape(hk, 2, wk, 2, Q)
            vol = (a[:, 0, :, 0] + a[:, 0, :, 1] + a[:, 1, :, 0] + a[:, 1, :, 1]) * 0.25

        xs, wx3 = starts[k], w3s[k]
        ys, wy3 = starts[4 + k], w3s[4 + k]

        ysh = ys >> 1
        par = (ys & 1).astype(jnp.bfloat16)[None, :]
        npar = 1.0 - par
        nm = hk // 2
        masks = [(ysh == m).astype(jnp.bfloat16)[None, :] for m in range(nm - 1)]
        E0 = jnp.zeros((wk, Q), jnp.bfloat16)
        O0 = jnp.zeros((wk, Q), jnp.bfloat16)
        E1 = jnp.zeros((wk, Q), jnp.bfloat16)
        O1 = jnp.zeros((wk, Q), jnp.bfloat16)
        for m in range(nm - 1):
            E0 = E0 + vol[2 * m] * masks[m]
            O0 = O0 + vol[2 * m + 1] * masks[m]
            E1 = E1 + vol[2 * m + 2] * masks[m]
            O1 = O1 + vol[2 * m + 3] * masks[m]
        rows = [E0 * npar + O0 * par,
                O0 * npar + E1 * par,
                E1 * npar + O1 * par]

        lxi = lax.broadcasted_iota(jnp.int32, (wk, Q), 0)
        xmasks = [(lxi == (xs + dx)[None, :]).astype(jnp.bfloat16)
                  for dx in range(3)]
        win = [[jnp.sum(rows[dy] * xmasks[dx], axis=0) for dx in range(3)]
               for dy in range(3)]

        winarr = jnp.stack([jnp.stack(w, axis=0) for w in win],
                           axis=0).astype(jnp.float32)
        t = jnp.sum(winarr[:, None, :, :] * wx3[None, :, :, :], axis=2)

        lvl = []
        for yo in range(-R, R + 1):
            a0, a1 = abs(yo), 9 - abs(yo)
            g = (wy3[yo + R, 0][None, :] * t[0, a0:a1]
                 + wy3[yo + R, 1][None, :] * t[1, a0:a1]
                 + wy3[yo + R, 2][None, :] * t[2, a0:a1])
            lvl.append(g)
        outs.append(jnp.concatenate(lvl, axis=0))

    out_ref[b] = jnp.stack(outs, axis=1)


@jax.jit
def kernel(feat1, feat2, curr_flow):
    f1 = feat1.reshape(B, C, Q).astype(jnp.bfloat16)
    f2t = feat2.reshape(B, C, Q).transpose(0, 2, 1).astype(jnp.bfloat16)
    flow = curr_flow.reshape(2, Q)

    out = pl.pallas_call(
        _lookup_body,
        out_shape=jax.ShapeDtypeStruct((B, L, 4, Q), jnp.float32),
    )(f2t, f1, flow)
    return out.reshape(B, L, 4, H8, W8)
